# baseline (speedup 1.0000x reference)
import functools

import jax
import jax.numpy as jnp
from jax import lax
from jax.experimental import pallas as pl
from jax.experimental.pallas import tpu as pltpu
from jax.experimental.pallas import tpu_sc as plsc

N = 10000
E = 320000
D = 128
H = 8
CPH = 16
G = 64
NCLS = 10
DW = 144

NC = 2
NS = 16
L = 16
NW = NC * NS
EPW = E // NW
K = 40
NCH = EPW // K
NPAD = 10240
RPS = NPAD // NS

_f32 = jnp.float32


def _tc_a_body(x_ref, w1_ref, as_ref, ex_ref, hs_ref, adp_ref, init_ref):
    xb = x_ref[...]
    h1 = jnp.dot(xb, w1_ref[...], preferred_element_type=_f32)
    sd = jnp.dot(h1, as_ref[...], preferred_element_type=_f32)
    hs_ref[...] = jnp.concatenate([h1, sd], axis=1)
    a_s = sd[:, 0:8]
    a_d = sd[:, 8:16]
    z = a_s + a_d
    z = jnp.where(z > 0, z, 0.2 * z)
    wself = jnp.exp(z)
    wse = jnp.dot(wself, ex_ref[...], preferred_element_type=_f32)
    zpad = jnp.zeros((h1.shape[0], 8), _f32)
    init_ref[...] = jnp.concatenate([h1 * wse, wself, zpad], axis=1)
    adp_ref[...] = jnp.concatenate([a_d, zpad], axis=1)


def _run_tc_a(x, W1, AS, EXP16):
    B = 1280
    grid = (NPAD // B,)
    return pl.pallas_call(
        _tc_a_body,
        grid=grid,
        in_specs=[
            pl.BlockSpec((B, D), lambda i: (i, 0)),
            pl.BlockSpec((D, D), lambda i: (0, 0)),
            pl.BlockSpec((D, 16), lambda i: (0, 0)),
            pl.BlockSpec((H, D), lambda i: (0, 0)),
        ],
        out_specs=[
            pl.BlockSpec((B, DW), lambda i: (i, 0)),
            pl.BlockSpec((B, 16), lambda i: (i, 0)),
            pl.BlockSpec((B, DW), lambda i: (i, 0)),
        ],
        out_shape=[
            jax.ShapeDtypeStruct((NPAD, DW), _f32),
            jax.ShapeDtypeStruct((NPAD, 16), _f32),
            jax.ShapeDtypeStruct((NPAD, DW), _f32),
        ],
    )(x, W1, AS, EXP16)


def _make_edge_kernel(lane_map):
    mesh = plsc.VectorSubcoreMesh(
        core_axis_name="c", subcore_axis_name="s", num_cores=NC, num_subcores=NS
    )

    @functools.partial(
        pl.kernel,
        out_type=jax.ShapeDtypeStruct((NC, NPAD, DW), _f32),
        mesh=mesh,
        compiler_params=pltpu.CompilerParams(use_tc_tiling_on_sc=False),
        scratch_types=[
            pltpu.VMEM((K,), jnp.int32),
            pltpu.VMEM((K,), jnp.int32),
            pltpu.VMEM((K,), jnp.int32),
            pltpu.VMEM((K,), jnp.int32),
            pltpu.VMEM((K,), jnp.int32),
            pltpu.VMEM((K,), jnp.int32),
            pltpu.VMEM((K, DW), _f32),
            pltpu.VMEM((K, DW), _f32),
            pltpu.VMEM((K, L), _f32),
            pltpu.VMEM((K, L), _f32),
            pltpu.VMEM((K, DW), _f32),
            pltpu.VMEM((K, DW), _f32),
            pltpu.VMEM_SHARED((NPAD, DW), _f32),
            pltpu.SemaphoreType.DMA,
            pltpu.SemaphoreType.DMA,
            pltpu.SemaphoreType.DMA,
            pltpu.SemaphoreType.DMA,
            pltpu.SemaphoreType.DMA,
            pltpu.SemaphoreType.DMA,
            pltpu.SemaphoreType.DMA,
            pltpu.SemaphoreType.DMA,
            pltpu.SemaphoreType.DMA,
            pltpu.SemaphoreType.DMA,
        ],
    )
    def edge_kernel(src_hbm, dst_hbm, hs_hbm, ad_hbm, init_hbm,
                    out_hbm, sv0, sv1, dg0, dg1, ds0, ds1, hsb0, hsb1, adb0,
                    adb1, ctb0, ctb1, acc, gsem0, gsem1, asem0,
                    asem1, ssem0, ssem1, isem0, isem1, jsem0, jsem1):
        cid = lax.axis_index("c")
        sid = lax.axis_index("s")
        wid = sid * NC + cid
        rbase = sid * RPS
        sv = (sv0, sv1)
        dg = (dg0, dg1)
        dvs = (ds0, ds1)
        hsb = (hsb0, hsb1)
        adb = (adb0, adb1)
        ctb = (ctb0, ctb1)
        gsem = (gsem0, gsem1)
        asem = (asem0, asem1)
        ssem = (ssem0, ssem1)
        isem = (isem0, isem1)
        jsem = (jsem0, jsem1)

        @pl.when(cid == 0)
        def _():
            pltpu.sync_copy(init_hbm.at[pl.ds(rbase, RPS)],
                            acc.at[pl.ds(rbase, RPS)])

        @pl.when(cid == 1)
        def _():
            def zrow(e, c):
                for t in range(DW // L):
                    ctb0[e, pl.ds(L * t, L)] = jnp.zeros((L,), _f32)
                return c

            lax.fori_loop(0, K, zrow, 0)

            def zcopy(t, c):
                off = pl.multiple_of(rbase + t * K, 8)
                pltpu.sync_copy(ctb0, acc.at[pl.ds(off, K)])
                return c

            lax.fori_loop(0, RPS // K, zcopy, 0)

        plsc.subcore_barrier()

        ebase = wid * EPW

        def fire_idx(ci, b):
            off = pl.multiple_of(ebase + ci * K, 8)
            pltpu.async_copy(src_hbm.at[pl.ds(off, K)], sv[b], isem[b])
            pltpu.async_copy(dst_hbm.at[pl.ds(off, K)], dg[b], isem[b])

        def wait_idx(ci, b):
            off = pl.multiple_of(ebase + ci * K, 8)
            pltpu.make_async_copy(src_hbm.at[pl.ds(off, K)], sv[b],
                                  isem[b]).wait()
            pltpu.make_async_copy(dst_hbm.at[pl.ds(off, K)], dg[b],
                                  isem[b]).wait()

        def fire_sidx(ci, b):
            off = pl.multiple_of(ebase + ci * K, 8)
            pltpu.async_copy(dst_hbm.at[pl.ds(off, K)], dvs[b], jsem[b])

        def wait_sidx(ci, b):
            off = pl.multiple_of(ebase + ci * K, 8)
            pltpu.make_async_copy(dst_hbm.at[pl.ds(off, K)], dvs[b],
                                  jsem[b]).wait()

        def fire_gathers(b):
            pltpu.async_copy(hs_hbm.at[sv[b]], hsb[b], gsem[b])
            pltpu.async_copy(ad_hbm.at[dg[b]], adb[b], asem[b])

        def wait_gathers(b):
            pltpu.make_async_copy(hs_hbm.at[sv[b]], hsb[b], gsem[b]).wait()
            pltpu.make_async_copy(ad_hbm.at[dg[b]], adb[b], asem[b]).wait()

        def fire_scatter(b):
            pltpu.async_copy(ctb[b], acc.at[dvs[b]], ssem[b], add=True)

        def wait_scatter(b):
            pltpu.make_async_copy(ctb[b], acc.at[dvs[b]], ssem[b]).wait()

        def compute(b):
            hsb_b = hsb[b]
            adb_b = adb[b]
            ctb_b = ctb[b]
            one_lane = len(set(lane_map)) == 1
            dnums = lax.GatherDimensionNumbers(
                offset_dims=(), collapsed_slice_dims=(0,),
                start_index_map=(0,))

            def bcast(w, lane):
                return lax.gather(
                    w, jnp.full((L, 1), lane, jnp.int32), dnums, (1,),
                    mode=lax.GatherScatterMode.PROMISE_IN_BOUNDS)

            @plsc.parallel_loop(0, K, 1, unroll=16)
            def _(e):
                asv = hsb_b[e, pl.ds(D, L)]
                adv = adb_b[e, :]
                z = asv + adv
                z = jnp.where(z > 0, z, 0.2 * z)
                w = jnp.exp(z)
                ctb_b[e, pl.ds(D, L)] = w
                if one_lane:
                    mult = bcast(w, lane_map[0])
                    for g in range(8):
                        ctb_b[e, pl.ds(CPH * g, L)] = (
                            hsb_b[e, pl.ds(CPH * g, L)] * mult)
                else:
                    for g in range(8):
                        mult = bcast(w, lane_map[g])
                        ctb_b[e, pl.ds(CPH * g, L)] = (
                            hsb_b[e, pl.ds(CPH * g, L)] * mult)

        fire_idx(0, 0)
        fire_idx(1, 1)
        fire_sidx(0, 0)
        wait_idx(0, 0)
        fire_gathers(0)

        def pair_body(j, carry):
            for b in range(2):
                i2 = 2 * j + b
                nb = 1 - b

                @pl.when(i2 <= NCH - 2)
                def _():
                    wait_idx(i2 + 1, nb)
                    fire_gathers(nb)

                wait_gathers(b)
                compute(b)

                @pl.when(i2 >= 1)
                def _():
                    wait_scatter(nb)

                @pl.when(i2 <= NCH - 2)
                def _():
                    fire_sidx(i2 + 1, nb)

                @pl.when(i2 <= NCH - 3)
                def _():
                    fire_idx(i2 + 2, b)

                wait_sidx(i2, b)
                fire_scatter(b)
            return carry

        lax.fori_loop(0, NCH // 2, pair_body, 0)
        wait_scatter(1)
        plsc.subcore_barrier()
        pltpu.sync_copy(acc.at[pl.ds(rbase, RPS)],
                        out_hbm.at[cid].at[pl.ds(rbase, RPS)])

    return edge_kernel


@functools.cache
def _edge_kernels():
    return _make_edge_kernel(tuple(range(8))), _make_edge_kernel((0,) * 8)


def _tc_c_body(p_ref, b1_ref, g_ref, be_ref, mu_ref, var_ref, w2_ref, a2_ref,
               ex_ref, hs2_ref, ad2p_ref, init2_ref):
    p = p_ref[...]
    tot = p[0] + p[1]
    num = tot[:, 0:D]
    den = tot[:, D:D + 8]
    dene = jnp.dot(den, ex_ref[...], preferred_element_type=_f32)
    out1 = num / dene + b1_ref[...]
    scale = g_ref[...] * lax.rsqrt(var_ref[...] + 1e-5)
    h = (out1 - mu_ref[...]) * scale + be_ref[...]
    h = jnp.where(h > 0, h, jnp.exp(h) - 1.0)
    h2 = jnp.dot(h, w2_ref[...], preferred_element_type=_f32)
    sd2 = jnp.dot(h2, a2_ref[...], preferred_element_type=_f32)
    hs2_ref[...] = jnp.concatenate([h2, sd2], axis=1)
    z = sd2[:, 0:1] + sd2[:, 1:2]
    z = jnp.where(z > 0, z, 0.2 * z)
    w2self = jnp.exp(z)
    zpad = jnp.zeros((h2.shape[0], 15), _f32)
    init2_ref[...] = jnp.concatenate([h2 * w2self, w2self, zpad], axis=1)
    ad2p_ref[...] = jnp.concatenate([sd2[:, 1:2], zpad], axis=1)


def _run_tc_c(part1, b1, bn_gamma, bn_beta, bn_mean, bn_var, W2, A2, EXP16):
    B = 1280
    grid = (NPAD // B,)
    row = lambda i: (0, 0)
    return pl.pallas_call(
        _tc_c_body,
        grid=grid,
        in_specs=[
            pl.BlockSpec((NC, B, DW), lambda i: (0, i, 0)),
            pl.BlockSpec((1, D), row),
            pl.BlockSpec((1, D), row),
            pl.BlockSpec((1, D), row),
            pl.BlockSpec((1, D), row),
            pl.BlockSpec((1, D), row),
            pl.BlockSpec((D, D), row),
            pl.BlockSpec((D, 16), row),
            pl.BlockSpec((H, D), row),
        ],
        out_specs=[
            pl.BlockSpec((B, DW), lambda i: (i, 0)),
            pl.BlockSpec((B, 16), lambda i: (i, 0)),
            pl.BlockSpec((B, DW), lambda i: (i, 0)),
        ],
        out_shape=[
            jax.ShapeDtypeStruct((NPAD, DW), _f32),
            jax.ShapeDtypeStruct((NPAD, 16), _f32),
            jax.ShapeDtypeStruct((NPAD, DW), _f32),
        ],
    )(part1, b1, bn_gamma, bn_beta, bn_mean, bn_var, W2, A2, EXP16)


def _tc_e_body(p_ref, b2_ref, batch_ref, f1w_ref, f1b_ref, f2w_ref, f2b_ref,
               out_ref, acc, cnt):
    i = pl.program_id(0)
    nsteps = pl.num_programs(0)

    @pl.when(i == 0)
    def _():
        acc[...] = jnp.zeros_like(acc)
        cnt[...] = jnp.zeros_like(cnt)

    p = p_ref[...]
    tot = p[0] + p[1]
    num = tot[:, 0:D]
    den = tot[:, D:D + 1]
    h2o = num / den + b2_ref[...]
    bb = batch_ref[...]
    Bn = h2o.shape[0]
    P = (bb == lax.broadcasted_iota(jnp.int32, (Bn, G), 1)).astype(_f32)
    dn = (((0,), (0,)), ((), ()))
    acc[...] += lax.dot_general(P, h2o, dn, preferred_element_type=_f32)
    cnt[...] += lax.dot_general(P, jnp.ones((Bn, D), _f32), dn,
                                preferred_element_type=_f32)

    @pl.when(i == nsteps - 1)
    def _():
        g = acc[...] / jnp.maximum(cnt[...], 1.0)
        g1 = jnp.dot(g, f1w_ref[...], preferred_element_type=_f32) + f1b_ref[...]
        g1 = jnp.where(g1 > 0, g1, jnp.exp(g1) - 1.0)
        out_ref[...] = (jnp.dot(g1, f2w_ref[...], preferred_element_type=_f32)
                        + f2b_ref[...])


def _run_tc_e(part2, b2, batch2d, fc1_W, fc1_b, fc2_W, fc2_b):
    B = 2000
    grid = (N // B,)
    row = lambda i: (0, 0)
    return pl.pallas_call(
        _tc_e_body,
        grid=grid,
        in_specs=[
            pl.BlockSpec((NC, B, DW), lambda i: (0, i, 0)),
            pl.BlockSpec((1, D), row),
            pl.BlockSpec((B, 1), lambda i: (i, 0)),
            pl.BlockSpec((D, G), row),
            pl.BlockSpec((1, G), row),
            pl.BlockSpec((G, NCLS), row),
            pl.BlockSpec((1, NCLS), row),
        ],
        out_specs=pl.BlockSpec((G, NCLS), row),
        out_shape=jax.ShapeDtypeStruct((G, NCLS), _f32),
        scratch_shapes=[
            pltpu.VMEM((G, D), _f32),
            pltpu.VMEM((G, D), _f32),
        ],
    )(part2, b2, batch2d, fc1_W, fc1_b, fc2_W, fc2_b)


def kernel(x, edge_index, batch, W1, a_src1, a_dst1, b1, bn_gamma, bn_beta,
           bn_mean, bn_var, W2, a_src2, a_dst2, b2, fc1_W, fc1_b, fc2_W,
           fc2_b):
    src = edge_index[0]
    dst = edge_index[1]

    eye = jnp.eye(H, dtype=_f32)
    As = (a_src1[:, :, None] * eye[:, None, :]).reshape(D, H)
    Ad = (a_dst1[:, :, None] * eye[:, None, :]).reshape(D, H)
    AS = jnp.concatenate([As, Ad], axis=1)
    EXP16 = jnp.repeat(jnp.eye(H, dtype=_f32), CPH, axis=1)
    A2 = jnp.concatenate(
        [a_src2.T, a_dst2.T, jnp.zeros((D, 14), _f32)], axis=1)
    edge_l1, edge_l2 = _edge_kernels()
    hs1, ad1p, init1 = _run_tc_a(x, W1, AS, EXP16)
    part1 = edge_l1(src, dst, hs1, ad1p, init1)
    hs2, ad2p, init2 = _run_tc_c(part1, b1.reshape(1, D),
                                 bn_gamma.reshape(1, D),
                                 bn_beta.reshape(1, D),
                                 bn_mean.reshape(1, D),
                                 bn_var.reshape(1, D), W2, A2, EXP16)
    part2 = edge_l2(src, dst, hs2, ad2p, init2)
    out = _run_tc_e(part2, b2.reshape(1, D), batch.reshape(N, 1),
                    fc1_W, fc1_b.reshape(1, G), fc2_W, fc2_b.reshape(1, NCLS))
    return out


# speedup vs baseline: 1.0427x; 1.0427x over previous
import functools

import jax
import jax.numpy as jnp
from jax import lax
from jax.experimental import pallas as pl
from jax.experimental.pallas import tpu as pltpu
from jax.experimental.pallas import tpu_sc as plsc

N = 10000
E = 320000
D = 128
H = 8
CPH = 16
G = 64
NCLS = 10
DW = 144

NC = 2
NS = 16
L = 16
NW = NC * NS
EPW = E // NW
K = 40
NCH = EPW // K
NPAD = 10240
RPS = NPAD // NS

_f32 = jnp.float32


def _tc_a_body(x_ref, w1_ref, as_ref, ex_ref, hs_ref, adp_ref, init_ref):
    xb = x_ref[...]
    h1 = jnp.dot(xb, w1_ref[...], preferred_element_type=_f32)
    sd = jnp.dot(h1, as_ref[...], preferred_element_type=_f32)
    hs_ref[...] = jnp.concatenate([h1, sd], axis=1)
    a_s = sd[:, 0:8]
    a_d = sd[:, 8:16]
    z = a_s + a_d
    z = jnp.where(z > 0, z, 0.2 * z)
    wself = jnp.exp(z)
    wse = jnp.dot(wself, ex_ref[...], preferred_element_type=_f32)
    zpad = jnp.zeros((h1.shape[0], 8), _f32)
    init_ref[...] = jnp.concatenate([h1 * wse, wself, zpad], axis=1)
    adp_ref[...] = jnp.concatenate([a_d, zpad], axis=1)


def _run_tc_a(x, W1, AS, EXP16):
    B = 1280
    grid = (NPAD // B,)
    return pl.pallas_call(
        _tc_a_body,
        grid=grid,
        in_specs=[
            pl.BlockSpec((B, D), lambda i: (i, 0)),
            pl.BlockSpec((D, D), lambda i: (0, 0)),
            pl.BlockSpec((D, 16), lambda i: (0, 0)),
            pl.BlockSpec((H, D), lambda i: (0, 0)),
        ],
        out_specs=[
            pl.BlockSpec((B, DW), lambda i: (i, 0)),
            pl.BlockSpec((B, 16), lambda i: (i, 0)),
            pl.BlockSpec((B, DW), lambda i: (i, 0)),
        ],
        out_shape=[
            jax.ShapeDtypeStruct((NPAD, DW), _f32),
            jax.ShapeDtypeStruct((NPAD, 16), _f32),
            jax.ShapeDtypeStruct((NPAD, DW), _f32),
        ],
    )(x, W1, AS, EXP16)


def _make_edge_kernel(lane_map):
    mesh = plsc.VectorSubcoreMesh(
        core_axis_name="c", subcore_axis_name="s", num_cores=NC, num_subcores=NS
    )

    @functools.partial(
        pl.kernel,
        out_type=jax.ShapeDtypeStruct((NC, NPAD, DW), _f32),
        mesh=mesh,
        compiler_params=pltpu.CompilerParams(use_tc_tiling_on_sc=False),
        scratch_types=[
            pltpu.VMEM((K,), jnp.int32),
            pltpu.VMEM((K,), jnp.int32),
            pltpu.VMEM((K,), jnp.int32),
            pltpu.VMEM((K,), jnp.int32),
            pltpu.VMEM((K,), jnp.int32),
            pltpu.VMEM((K,), jnp.int32),
            pltpu.VMEM((K, DW), _f32),
            pltpu.VMEM((K, DW), _f32),
            pltpu.VMEM((K, L), _f32),
            pltpu.VMEM((K, L), _f32),
            pltpu.VMEM((K, DW), _f32),
            pltpu.VMEM((K, DW), _f32),
            pltpu.VMEM_SHARED((NPAD, DW), _f32),
            pltpu.SemaphoreType.DMA,
            pltpu.SemaphoreType.DMA,
            pltpu.SemaphoreType.DMA,
            pltpu.SemaphoreType.DMA,
            pltpu.SemaphoreType.DMA,
            pltpu.SemaphoreType.DMA,
            pltpu.SemaphoreType.DMA,
            pltpu.SemaphoreType.DMA,
            pltpu.SemaphoreType.DMA,
            pltpu.SemaphoreType.DMA,
        ],
    )
    def edge_kernel(src_hbm, dst_hbm, hs_hbm, ad_hbm, init_hbm,
                    out_hbm, sv0, sv1, dg0, dg1, ds0, ds1, hsb0, hsb1, adb0,
                    adb1, ctb0, ctb1, acc, gsem0, gsem1, asem0,
                    asem1, ssem0, ssem1, isem0, isem1, jsem0, jsem1):
        cid = lax.axis_index("c")
        sid = lax.axis_index("s")
        wid = sid * NC + cid
        rbase = sid * RPS
        sv = (sv0, sv1)
        dg = (dg0, dg1)
        dvs = (ds0, ds1)
        hsb = (hsb0, hsb1)
        adb = (adb0, adb1)
        ctb = (ctb0, ctb1)
        gsem = (gsem0, gsem1)
        asem = (asem0, asem1)
        ssem = (ssem0, ssem1)
        isem = (isem0, isem1)
        jsem = (jsem0, jsem1)

        @pl.when(cid == 0)
        def _():
            pltpu.sync_copy(init_hbm.at[pl.ds(rbase, RPS)],
                            acc.at[pl.ds(rbase, RPS)])

        @pl.when(cid == 1)
        def _():
            def zrow(e, c):
                for t in range(DW // L):
                    ctb0[e, pl.ds(L * t, L)] = jnp.zeros((L,), _f32)
                return c

            lax.fori_loop(0, K, zrow, 0)

            def zcopy(t, c):
                off = pl.multiple_of(rbase + t * K, 8)
                pltpu.sync_copy(ctb0, acc.at[pl.ds(off, K)])
                return c

            lax.fori_loop(0, RPS // K, zcopy, 0)

        plsc.subcore_barrier()

        ebase = wid * EPW

        def fire_idx(ci, b):
            off = pl.multiple_of(ebase + ci * K, 8)
            pltpu.async_copy(src_hbm.at[pl.ds(off, K)], sv[b], isem[b])
            pltpu.async_copy(dst_hbm.at[pl.ds(off, K)], dg[b], isem[b])

        def wait_idx(ci, b):
            off = pl.multiple_of(ebase + ci * K, 8)
            pltpu.make_async_copy(src_hbm.at[pl.ds(off, K)], sv[b],
                                  isem[b]).wait()
            pltpu.make_async_copy(dst_hbm.at[pl.ds(off, K)], dg[b],
                                  isem[b]).wait()

        def fire_sidx(ci, b):
            off = pl.multiple_of(ebase + ci * K, 8)
            pltpu.async_copy(dst_hbm.at[pl.ds(off, K)], dvs[b], jsem[b])

        def wait_sidx(ci, b):
            off = pl.multiple_of(ebase + ci * K, 8)
            pltpu.make_async_copy(dst_hbm.at[pl.ds(off, K)], dvs[b],
                                  jsem[b]).wait()

        def fire_gathers(b):
            pltpu.async_copy(hs_hbm.at[sv[b]], hsb[b], gsem[b])
            pltpu.async_copy(ad_hbm.at[dg[b]], adb[b], asem[b])

        def wait_gathers(b):
            pltpu.make_async_copy(hs_hbm.at[sv[b]], hsb[b], gsem[b]).wait()
            pltpu.make_async_copy(ad_hbm.at[dg[b]], adb[b], asem[b]).wait()

        def fire_scatter(b):
            pltpu.async_copy(ctb[b], acc.at[dvs[b]], ssem[b], add=True)

        def wait_scatter(b):
            pltpu.make_async_copy(ctb[b], acc.at[dvs[b]], ssem[b]).wait()

        def compute(b):
            hsb_b = hsb[b]
            adb_b = adb[b]
            ctb_b = ctb[b]
            one_lane = len(set(lane_map)) == 1
            dnums = lax.GatherDimensionNumbers(
                offset_dims=(), collapsed_slice_dims=(0,),
                start_index_map=(0,))

            def bcast(w, lane):
                return lax.gather(
                    w, jnp.full((L, 1), lane, jnp.int32), dnums, (1,),
                    mode=lax.GatherScatterMode.PROMISE_IN_BOUNDS)

            @plsc.parallel_loop(0, K, 1, unroll=4)
            def _(e):
                asv = hsb_b[e, pl.ds(D, L)]
                adv = adb_b[e, :]
                z = asv + adv
                z = jnp.where(z > 0, z, 0.2 * z)
                w = jnp.exp(z)
                ctb_b[e, pl.ds(D, L)] = w
                if one_lane:
                    mult = bcast(w, lane_map[0])
                    for g in range(8):
                        ctb_b[e, pl.ds(CPH * g, L)] = (
                            hsb_b[e, pl.ds(CPH * g, L)] * mult)
                else:
                    for g in range(8):
                        mult = bcast(w, lane_map[g])
                        ctb_b[e, pl.ds(CPH * g, L)] = (
                            hsb_b[e, pl.ds(CPH * g, L)] * mult)

        fire_idx(0, 0)
        fire_idx(1, 1)
        fire_sidx(0, 0)
        wait_idx(0, 0)
        fire_gathers(0)

        def pair_body(j, carry):
            for b in range(2):
                i2 = 2 * j + b
                nb = 1 - b

                @pl.when(i2 <= NCH - 2)
                def _():
                    wait_idx(i2 + 1, nb)
                    fire_gathers(nb)

                wait_gathers(b)
                compute(b)

                @pl.when(i2 >= 1)
                def _():
                    wait_scatter(nb)

                @pl.when(i2 <= NCH - 2)
                def _():
                    fire_sidx(i2 + 1, nb)

                @pl.when(i2 <= NCH - 3)
                def _():
                    fire_idx(i2 + 2, b)

                wait_sidx(i2, b)
                fire_scatter(b)
            return carry

        lax.fori_loop(0, NCH // 2, pair_body, 0)
        wait_scatter(1)
        plsc.subcore_barrier()
        pltpu.sync_copy(acc.at[pl.ds(rbase, RPS)],
                        out_hbm.at[cid].at[pl.ds(rbase, RPS)])

    return edge_kernel


@functools.cache
def _edge_kernels():
    return _make_edge_kernel(tuple(range(8))), _make_edge_kernel((0,) * 8)


def _tc_c_body(p_ref, b1_ref, g_ref, be_ref, mu_ref, var_ref, w2_ref, a2_ref,
               ex_ref, hs2_ref, ad2p_ref, init2_ref):
    p = p_ref[...]
    tot = p[0] + p[1]
    num = tot[:, 0:D]
    den = tot[:, D:D + 8]
    dene = jnp.dot(den, ex_ref[...], preferred_element_type=_f32)
    out1 = num / dene + b1_ref[...]
    scale = g_ref[...] * lax.rsqrt(var_ref[...] + 1e-5)
    h = (out1 - mu_ref[...]) * scale + be_ref[...]
    h = jnp.where(h > 0, h, jnp.exp(h) - 1.0)
    h2 = jnp.dot(h, w2_ref[...], preferred_element_type=_f32)
    sd2 = jnp.dot(h2, a2_ref[...], preferred_element_type=_f32)
    hs2_ref[...] = jnp.concatenate([h2, sd2], axis=1)
    z = sd2[:, 0:1] + sd2[:, 1:2]
    z = jnp.where(z > 0, z, 0.2 * z)
    w2self = jnp.exp(z)
    zpad = jnp.zeros((h2.shape[0], 15), _f32)
    init2_ref[...] = jnp.concatenate([h2 * w2self, w2self, zpad], axis=1)
    ad2p_ref[...] = jnp.concatenate([sd2[:, 1:2], zpad], axis=1)


def _run_tc_c(part1, b1, bn_gamma, bn_beta, bn_mean, bn_var, W2, A2, EXP16):
    B = 1280
    grid = (NPAD // B,)
    row = lambda i: (0, 0)
    return pl.pallas_call(
        _tc_c_body,
        grid=grid,
        in_specs=[
            pl.BlockSpec((NC, B, DW), lambda i: (0, i, 0)),
            pl.BlockSpec((1, D), row),
            pl.BlockSpec((1, D), row),
            pl.BlockSpec((1, D), row),
            pl.BlockSpec((1, D), row),
            pl.BlockSpec((1, D), row),
            pl.BlockSpec((D, D), row),
            pl.BlockSpec((D, 16), row),
            pl.BlockSpec((H, D), row),
        ],
        out_specs=[
            pl.BlockSpec((B, DW), lambda i: (i, 0)),
            pl.BlockSpec((B, 16), lambda i: (i, 0)),
            pl.BlockSpec((B, DW), lambda i: (i, 0)),
        ],
        out_shape=[
            jax.ShapeDtypeStruct((NPAD, DW), _f32),
            jax.ShapeDtypeStruct((NPAD, 16), _f32),
            jax.ShapeDtypeStruct((NPAD, DW), _f32),
        ],
    )(part1, b1, bn_gamma, bn_beta, bn_mean, bn_var, W2, A2, EXP16)


def _tc_e_body(p_ref, b2_ref, batch_ref, f1w_ref, f1b_ref, f2w_ref, f2b_ref,
               out_ref, acc, cnt):
    i = pl.program_id(0)
    nsteps = pl.num_programs(0)

    @pl.when(i == 0)
    def _():
        acc[...] = jnp.zeros_like(acc)
        cnt[...] = jnp.zeros_like(cnt)

    p = p_ref[...]
    tot = p[0] + p[1]
    num = tot[:, 0:D]
    den = tot[:, D:D + 1]
    h2o = num / den + b2_ref[...]
    bb = batch_ref[...]
    Bn = h2o.shape[0]
    P = (bb == lax.broadcasted_iota(jnp.int32, (Bn, G), 1)).astype(_f32)
    dn = (((0,), (0,)), ((), ()))
    acc[...] += lax.dot_general(P, h2o, dn, preferred_element_type=_f32)
    cnt[...] += lax.dot_general(P, jnp.ones((Bn, D), _f32), dn,
                                preferred_element_type=_f32)

    @pl.when(i == nsteps - 1)
    def _():
        g = acc[...] / jnp.maximum(cnt[...], 1.0)
        g1 = jnp.dot(g, f1w_ref[...], preferred_element_type=_f32) + f1b_ref[...]
        g1 = jnp.where(g1 > 0, g1, jnp.exp(g1) - 1.0)
        out_ref[...] = (jnp.dot(g1, f2w_ref[...], preferred_element_type=_f32)
                        + f2b_ref[...])


def _run_tc_e(part2, b2, batch2d, fc1_W, fc1_b, fc2_W, fc2_b):
    B = 2000
    grid = (N // B,)
    row = lambda i: (0, 0)
    return pl.pallas_call(
        _tc_e_body,
        grid=grid,
        in_specs=[
            pl.BlockSpec((NC, B, DW), lambda i: (0, i, 0)),
            pl.BlockSpec((1, D), row),
            pl.BlockSpec((B, 1), lambda i: (i, 0)),
            pl.BlockSpec((D, G), row),
            pl.BlockSpec((1, G), row),
            pl.BlockSpec((G, NCLS), row),
            pl.BlockSpec((1, NCLS), row),
        ],
        out_specs=pl.BlockSpec((G, NCLS), row),
        out_shape=jax.ShapeDtypeStruct((G, NCLS), _f32),
        scratch_shapes=[
            pltpu.VMEM((G, D), _f32),
            pltpu.VMEM((G, D), _f32),
        ],
    )(part2, b2, batch2d, fc1_W, fc1_b, fc2_W, fc2_b)


def kernel(x, edge_index, batch, W1, a_src1, a_dst1, b1, bn_gamma, bn_beta,
           bn_mean, bn_var, W2, a_src2, a_dst2, b2, fc1_W, fc1_b, fc2_W,
           fc2_b):
    src = edge_index[0]
    dst = edge_index[1]

    eye = jnp.eye(H, dtype=_f32)
    As = (a_src1[:, :, None] * eye[:, None, :]).reshape(D, H)
    Ad = (a_dst1[:, :, None] * eye[:, None, :]).reshape(D, H)
    AS = jnp.concatenate([As, Ad], axis=1)
    EXP16 = jnp.repeat(jnp.eye(H, dtype=_f32), CPH, axis=1)
    A2 = jnp.concatenate(
        [a_src2.T, a_dst2.T, jnp.zeros((D, 14), _f32)], axis=1)
    edge_l1, edge_l2 = _edge_kernels()
    hs1, ad1p, init1 = _run_tc_a(x, W1, AS, EXP16)
    part1 = edge_l1(src, dst, hs1, ad1p, init1)
    hs2, ad2p, init2 = _run_tc_c(part1, b1.reshape(1, D),
                                 bn_gamma.reshape(1, D),
                                 bn_beta.reshape(1, D),
                                 bn_mean.reshape(1, D),
                                 bn_var.reshape(1, D), W2, A2, EXP16)
    part2 = edge_l2(src, dst, hs2, ad2p, init2)
    out = _run_tc_e(part2, b2.reshape(1, D), batch.reshape(N, 1),
                    fc1_W, fc1_b.reshape(1, G), fc2_W, fc2_b.reshape(1, NCLS))
    return out


# speedup vs baseline: 1.0431x; 1.0004x over previous
import functools

import jax
import jax.numpy as jnp
from jax import lax
from jax.experimental import pallas as pl
from jax.experimental.pallas import tpu as pltpu
from jax.experimental.pallas import tpu_sc as plsc

N = 10000
E = 320000
D = 128
H = 8
CPH = 16
G = 64
NCLS = 10
DW = 144

NC = 2
NS = 16
L = 16
NW = NC * NS
EPW = E // NW
K = 40
NCH = EPW // K
NPAD = 10240
RPS = NPAD // NS

_f32 = jnp.float32


def _tc_a_body(x_ref, w1_ref, as_ref, ex_ref, hs_ref, adp_ref, init_ref):
    xb = x_ref[...]
    h1 = jnp.dot(xb, w1_ref[...], preferred_element_type=_f32)
    sd = jnp.dot(h1, as_ref[...], preferred_element_type=_f32)
    hs_ref[...] = jnp.concatenate([h1, sd], axis=1)
    a_s = sd[:, 0:8]
    a_d = sd[:, 8:16]
    z = a_s + a_d
    z = jnp.where(z > 0, z, 0.2 * z)
    wself = jnp.exp(z)
    wse = jnp.dot(wself, ex_ref[...], preferred_element_type=_f32)
    zpad = jnp.zeros((h1.shape[0], 8), _f32)
    init_ref[...] = jnp.concatenate([h1 * wse, wself, zpad], axis=1)
    adp_ref[...] = jnp.concatenate([a_d, zpad], axis=1)


def _run_tc_a(x, W1, AS, EXP16):
    B = 1280
    grid = (NPAD // B,)
    return pl.pallas_call(
        _tc_a_body,
        grid=grid,
        in_specs=[
            pl.BlockSpec((B, D), lambda i: (i, 0)),
            pl.BlockSpec((D, D), lambda i: (0, 0)),
            pl.BlockSpec((D, 16), lambda i: (0, 0)),
            pl.BlockSpec((H, D), lambda i: (0, 0)),
        ],
        out_specs=[
            pl.BlockSpec((B, DW), lambda i: (i, 0)),
            pl.BlockSpec((B, 16), lambda i: (i, 0)),
            pl.BlockSpec((B, DW), lambda i: (i, 0)),
        ],
        out_shape=[
            jax.ShapeDtypeStruct((NPAD, DW), _f32),
            jax.ShapeDtypeStruct((NPAD, 16), _f32),
            jax.ShapeDtypeStruct((NPAD, DW), _f32),
        ],
    )(x, W1, AS, EXP16)


def _make_edge_kernel(lane_map):
    mesh = plsc.VectorSubcoreMesh(
        core_axis_name="c", subcore_axis_name="s", num_cores=NC, num_subcores=NS
    )

    @functools.partial(
        pl.kernel,
        out_type=jax.ShapeDtypeStruct((NC, NPAD, DW), _f32),
        mesh=mesh,
        compiler_params=pltpu.CompilerParams(use_tc_tiling_on_sc=False),
        scratch_types=[
            pltpu.VMEM((K,), jnp.int32),
            pltpu.VMEM((K,), jnp.int32),
            pltpu.VMEM((K,), jnp.int32),
            pltpu.VMEM((K,), jnp.int32),
            pltpu.VMEM((K,), jnp.int32),
            pltpu.VMEM((K,), jnp.int32),
            pltpu.VMEM((K, DW), _f32),
            pltpu.VMEM((K, DW), _f32),
            pltpu.VMEM((K, L), _f32),
            pltpu.VMEM((K, L), _f32),
            pltpu.VMEM((K, DW), _f32),
            pltpu.VMEM((K, DW), _f32),
            pltpu.VMEM_SHARED((NPAD, DW), _f32),
            pltpu.SemaphoreType.DMA,
            pltpu.SemaphoreType.DMA,
            pltpu.SemaphoreType.DMA,
            pltpu.SemaphoreType.DMA,
            pltpu.SemaphoreType.DMA,
            pltpu.SemaphoreType.DMA,
            pltpu.SemaphoreType.DMA,
            pltpu.SemaphoreType.DMA,
            pltpu.SemaphoreType.DMA,
            pltpu.SemaphoreType.DMA,
        ],
    )
    def edge_kernel(src_hbm, dst_hbm, hs_hbm, ad_hbm, init_hbm,
                    out_hbm, sv0, sv1, dg0, dg1, ds0, ds1, hsb0, hsb1, adb0,
                    adb1, ctb0, ctb1, acc, gsem0, gsem1, asem0,
                    asem1, ssem0, ssem1, isem0, isem1, jsem0, jsem1):
        cid = lax.axis_index("c")
        sid = lax.axis_index("s")
        wid = sid * NC + cid
        rbase = sid * RPS
        sv = (sv0, sv1)
        dg = (dg0, dg1)
        dvs = (ds0, ds1)
        hsb = (hsb0, hsb1)
        adb = (adb0, adb1)
        ctb = (ctb0, ctb1)
        gsem = (gsem0, gsem1)
        asem = (asem0, asem1)
        ssem = (ssem0, ssem1)
        isem = (isem0, isem1)
        jsem = (jsem0, jsem1)

        @pl.when(cid == 0)
        def _():
            pltpu.sync_copy(init_hbm.at[pl.ds(rbase, RPS)],
                            acc.at[pl.ds(rbase, RPS)])

        @pl.when(cid == 1)
        def _():
            def zrow(e, c):
                for t in range(DW // L):
                    ctb0[e, pl.ds(L * t, L)] = jnp.zeros((L,), _f32)
                return c

            lax.fori_loop(0, K, zrow, 0)

            def zcopy(t, c):
                off = pl.multiple_of(rbase + t * K, 8)
                pltpu.sync_copy(ctb0, acc.at[pl.ds(off, K)])
                return c

            lax.fori_loop(0, RPS // K, zcopy, 0)

        plsc.subcore_barrier()

        ebase = wid * EPW

        def fire_idx(ci, b):
            off = pl.multiple_of(ebase + ci * K, 8)
            pltpu.async_copy(src_hbm.at[pl.ds(off, K)], sv[b], isem[b])
            pltpu.async_copy(dst_hbm.at[pl.ds(off, K)], dg[b], isem[b])

        def wait_idx(ci, b):
            off = pl.multiple_of(ebase + ci * K, 8)
            pltpu.make_async_copy(src_hbm.at[pl.ds(off, K)], sv[b],
                                  isem[b]).wait()
            pltpu.make_async_copy(dst_hbm.at[pl.ds(off, K)], dg[b],
                                  isem[b]).wait()

        def fire_sidx(ci, b):
            off = pl.multiple_of(ebase + ci * K, 8)
            pltpu.async_copy(dst_hbm.at[pl.ds(off, K)], dvs[b], jsem[b])

        def wait_sidx(ci, b):
            off = pl.multiple_of(ebase + ci * K, 8)
            pltpu.make_async_copy(dst_hbm.at[pl.ds(off, K)], dvs[b],
                                  jsem[b]).wait()

        def fire_gathers(b):
            pltpu.async_copy(hs_hbm.at[sv[b]], hsb[b], gsem[b])
            pltpu.async_copy(ad_hbm.at[dg[b]], adb[b], asem[b])

        def wait_gathers(b):
            pltpu.make_async_copy(hs_hbm.at[sv[b]], hsb[b], gsem[b]).wait()
            pltpu.make_async_copy(ad_hbm.at[dg[b]], adb[b], asem[b]).wait()

        def fire_scatter(b):
            pltpu.async_copy(ctb[b], acc.at[dvs[b]], ssem[b], add=True)

        def wait_scatter(b):
            pltpu.make_async_copy(ctb[b], acc.at[dvs[b]], ssem[b]).wait()

        def compute(b):
            hsb_b = hsb[b]
            adb_b = adb[b]
            ctb_b = ctb[b]
            one_lane = len(set(lane_map)) == 1
            dnums = lax.GatherDimensionNumbers(
                offset_dims=(), collapsed_slice_dims=(0,),
                start_index_map=(0,))

            def bcast(w, lane):
                return lax.gather(
                    w, jnp.full((L, 1), lane, jnp.int32), dnums, (1,),
                    mode=lax.GatherScatterMode.PROMISE_IN_BOUNDS)

            @plsc.parallel_loop(0, K, 1, unroll=8)
            def _(e):
                asv = hsb_b[e, pl.ds(D, L)]
                adv = adb_b[e, :]
                z = asv + adv
                z = jnp.where(z > 0, z, 0.2 * z)
                w = jnp.exp(z)
                ctb_b[e, pl.ds(D, L)] = w
                if one_lane:
                    mult = bcast(w, lane_map[0])
                    for g in range(8):
                        ctb_b[e, pl.ds(CPH * g, L)] = (
                            hsb_b[e, pl.ds(CPH * g, L)] * mult)
                else:
                    for g in range(8):
                        mult = bcast(w, lane_map[g])
                        ctb_b[e, pl.ds(CPH * g, L)] = (
                            hsb_b[e, pl.ds(CPH * g, L)] * mult)

        fire_idx(0, 0)
        fire_idx(1, 1)
        fire_sidx(0, 0)
        wait_idx(0, 0)
        fire_gathers(0)

        def pair_body(j, carry):
            for b in range(2):
                i2 = 2 * j + b
                nb = 1 - b

                @pl.when(i2 <= NCH - 2)
                def _():
                    wait_idx(i2 + 1, nb)
                    fire_gathers(nb)

                wait_gathers(b)
                compute(b)

                @pl.when(i2 >= 1)
                def _():
                    wait_scatter(nb)

                @pl.when(i2 <= NCH - 2)
                def _():
                    fire_sidx(i2 + 1, nb)

                @pl.when(i2 <= NCH - 3)
                def _():
                    fire_idx(i2 + 2, b)

                wait_sidx(i2, b)
                fire_scatter(b)
            return carry

        lax.fori_loop(0, NCH // 2, pair_body, 0)
        wait_scatter(1)
        plsc.subcore_barrier()
        pltpu.sync_copy(acc.at[pl.ds(rbase, RPS)],
                        out_hbm.at[cid].at[pl.ds(rbase, RPS)])

    return edge_kernel


@functools.cache
def _edge_kernels():
    return _make_edge_kernel(tuple(range(8))), _make_edge_kernel((0,) * 8)


def _tc_c_body(p_ref, b1_ref, g_ref, be_ref, mu_ref, var_ref, w2_ref, a2_ref,
               ex_ref, hs2_ref, ad2p_ref, init2_ref):
    p = p_ref[...]
    tot = p[0] + p[1]
    num = tot[:, 0:D]
    den = tot[:, D:D + 8]
    dene = jnp.dot(den, ex_ref[...], preferred_element_type=_f32)
    out1 = num / dene + b1_ref[...]
    scale = g_ref[...] * lax.rsqrt(var_ref[...] + 1e-5)
    h = (out1 - mu_ref[...]) * scale + be_ref[...]
    h = jnp.where(h > 0, h, jnp.exp(h) - 1.0)
    h2 = jnp.dot(h, w2_ref[...], preferred_element_type=_f32)
    sd2 = jnp.dot(h2, a2_ref[...], preferred_element_type=_f32)
    hs2_ref[...] = jnp.concatenate([h2, sd2], axis=1)
    z = sd2[:, 0:1] + sd2[:, 1:2]
    z = jnp.where(z > 0, z, 0.2 * z)
    w2self = jnp.exp(z)
    zpad = jnp.zeros((h2.shape[0], 15), _f32)
    init2_ref[...] = jnp.concatenate([h2 * w2self, w2self, zpad], axis=1)
    ad2p_ref[...] = jnp.concatenate([sd2[:, 1:2], zpad], axis=1)


def _run_tc_c(part1, b1, bn_gamma, bn_beta, bn_mean, bn_var, W2, A2, EXP16):
    B = 1280
    grid = (NPAD // B,)
    row = lambda i: (0, 0)
    return pl.pallas_call(
        _tc_c_body,
        grid=grid,
        in_specs=[
            pl.BlockSpec((NC, B, DW), lambda i: (0, i, 0)),
            pl.BlockSpec((1, D), row),
            pl.BlockSpec((1, D), row),
            pl.BlockSpec((1, D), row),
            pl.BlockSpec((1, D), row),
            pl.BlockSpec((1, D), row),
            pl.BlockSpec((D, D), row),
            pl.BlockSpec((D, 16), row),
            pl.BlockSpec((H, D), row),
        ],
        out_specs=[
            pl.BlockSpec((B, DW), lambda i: (i, 0)),
            pl.BlockSpec((B, 16), lambda i: (i, 0)),
            pl.BlockSpec((B, DW), lambda i: (i, 0)),
        ],
        out_shape=[
            jax.ShapeDtypeStruct((NPAD, DW), _f32),
            jax.ShapeDtypeStruct((NPAD, 16), _f32),
            jax.ShapeDtypeStruct((NPAD, DW), _f32),
        ],
    )(part1, b1, bn_gamma, bn_beta, bn_mean, bn_var, W2, A2, EXP16)


def _tc_e_body(p_ref, b2_ref, batch_ref, f1w_ref, f1b_ref, f2w_ref, f2b_ref,
               out_ref, acc, cnt):
    i = pl.program_id(0)
    nsteps = pl.num_programs(0)

    @pl.when(i == 0)
    def _():
        acc[...] = jnp.zeros_like(acc)
        cnt[...] = jnp.zeros_like(cnt)

    p = p_ref[...]
    tot = p[0] + p[1]
    num = tot[:, 0:D]
    den = tot[:, D:D + 1]
    h2o = num / den + b2_ref[...]
    bb = batch_ref[...]
    Bn = h2o.shape[0]
    P = (bb == lax.broadcasted_iota(jnp.int32, (Bn, G), 1)).astype(_f32)
    dn = (((0,), (0,)), ((), ()))
    acc[...] += lax.dot_general(P, h2o, dn, preferred_element_type=_f32)
    cnt[...] += lax.dot_general(P, jnp.ones((Bn, D), _f32), dn,
                                preferred_element_type=_f32)

    @pl.when(i == nsteps - 1)
    def _():
        g = acc[...] / jnp.maximum(cnt[...], 1.0)
        g1 = jnp.dot(g, f1w_ref[...], preferred_element_type=_f32) + f1b_ref[...]
        g1 = jnp.where(g1 > 0, g1, jnp.exp(g1) - 1.0)
        out_ref[...] = (jnp.dot(g1, f2w_ref[...], preferred_element_type=_f32)
                        + f2b_ref[...])


def _run_tc_e(part2, b2, batch2d, fc1_W, fc1_b, fc2_W, fc2_b):
    B = 2000
    grid = (N // B,)
    row = lambda i: (0, 0)
    return pl.pallas_call(
        _tc_e_body,
        grid=grid,
        in_specs=[
            pl.BlockSpec((NC, B, DW), lambda i: (0, i, 0)),
            pl.BlockSpec((1, D), row),
            pl.BlockSpec((B, 1), lambda i: (i, 0)),
            pl.BlockSpec((D, G), row),
            pl.BlockSpec((1, G), row),
            pl.BlockSpec((G, NCLS), row),
            pl.BlockSpec((1, NCLS), row),
        ],
        out_specs=pl.BlockSpec((G, NCLS), row),
        out_shape=jax.ShapeDtypeStruct((G, NCLS), _f32),
        scratch_shapes=[
            pltpu.VMEM((G, D), _f32),
            pltpu.VMEM((G, D), _f32),
        ],
    )(part2, b2, batch2d, fc1_W, fc1_b, fc2_W, fc2_b)


def kernel(x, edge_index, batch, W1, a_src1, a_dst1, b1, bn_gamma, bn_beta,
           bn_mean, bn_var, W2, a_src2, a_dst2, b2, fc1_W, fc1_b, fc2_W,
           fc2_b):
    src = edge_index[0]
    dst = edge_index[1]

    eye = jnp.eye(H, dtype=_f32)
    As = (a_src1[:, :, None] * eye[:, None, :]).reshape(D, H)
    Ad = (a_dst1[:, :, None] * eye[:, None, :]).reshape(D, H)
    AS = jnp.concatenate([As, Ad], axis=1)
    EXP16 = jnp.repeat(jnp.eye(H, dtype=_f32), CPH, axis=1)
    A2 = jnp.concatenate(
        [a_src2.T, a_dst2.T, jnp.zeros((D, 14), _f32)], axis=1)
    edge_l1, edge_l2 = _edge_kernels()
    hs1, ad1p, init1 = _run_tc_a(x, W1, AS, EXP16)
    part1 = edge_l1(src, dst, hs1, ad1p, init1)
    hs2, ad2p, init2 = _run_tc_c(part1, b1.reshape(1, D),
                                 bn_gamma.reshape(1, D),
                                 bn_beta.reshape(1, D),
                                 bn_mean.reshape(1, D),
                                 bn_var.reshape(1, D), W2, A2, EXP16)
    part2 = edge_l2(src, dst, hs2, ad2p, init2)
    out = _run_tc_e(part2, b2.reshape(1, D), batch.reshape(N, 1),
                    fc1_W, fc1_b.reshape(1, G), fc2_W, fc2_b.reshape(1, NCLS))
    return out
